# baseline (speedup 1.0000x reference)
import functools

import jax
import jax.numpy as jnp
from jax import lax
from jax.experimental import pallas as pl
from jax.experimental.pallas import tpu as pltpu
from jax.experimental.pallas import tpu_sc as plsc

_NC = 2
_NS = 16
_R = 32


def _sc_add_kernel(T, x_hbm, pos_hbm, out_hbm, xbuf, posbuf):
    B = x_hbm.shape[0]
    D = x_hbm.shape[2]
    nw = _NC * _NS
    strip = T // nw
    wid = lax.axis_index("s") * _NC + lax.axis_index("c")
    t0 = wid * strip
    nslice = D // 16

    def chunk_body(c, _):
        row0 = t0 + c * _R
        pltpu.sync_copy(pos_hbm.at[pl.ds(row0, _R), :], posbuf)
        for b in range(B):
            pltpu.sync_copy(x_hbm.at[b, pl.ds(row0, _R), :], xbuf)

            def row_body(r, _):
                for j in range(nslice):
                    v = posbuf[r, pl.ds(j * 16, 16)]
                    plsc.addupdate(xbuf.at[r, pl.ds(j * 16, 16)], v)
                return 0

            lax.fori_loop(0, _R, row_body, 0)
            pltpu.sync_copy(xbuf, out_hbm.at[b, pl.ds(row0, _R), :])
        return 0

    lax.fori_loop(0, strip // _R, chunk_body, 0)


def kernel(x, pos_embedding):
    B, T, D = x.shape
    mesh = plsc.VectorSubcoreMesh(core_axis_name="c", subcore_axis_name="s")
    f = functools.partial(
        pl.kernel,
        mesh=mesh,
        out_type=jax.ShapeDtypeStruct((B, T, D), x.dtype),
        scratch_types=[
            pltpu.VMEM((_R, D), jnp.float32),
            pltpu.VMEM((_R, D), jnp.float32),
        ],
    )(functools.partial(_sc_add_kernel, T))
    return f(x, pos_embedding)


# speedup vs baseline: 2.3226x; 2.3226x over previous
import functools

import jax
import jax.numpy as jnp
from jax import lax
from jax.experimental import pallas as pl
from jax.experimental.pallas import tpu as pltpu
from jax.experimental.pallas import tpu_sc as plsc

_NC = 2
_NS = 16
_R = 32


def _sc_add_kernel(T, x_hbm, pos_hbm, out_hbm, xb0, xb1, posbuf,
                   in_s0, in_s1, out_s0, out_s1):
    B = x_hbm.shape[0]
    D = x_hbm.shape[2]
    nw = _NC * _NS
    strip = T // nw
    wid = lax.axis_index("s") * _NC + lax.axis_index("c")
    t0 = wid * strip
    nslice = D // 16
    nchunk = strip // _R
    xbufs = (xb0, xb1)
    in_sems = (in_s0, in_s1)
    out_sems = (out_s0, out_s1)

    units = [(c, b) for c in range(nchunk) for b in range(B)]
    n_units = len(units)

    def start_in(u):
        c, b = units[u]
        k = u % 2
        return pltpu.async_copy(
            x_hbm.at[b, pl.ds(t0 + c * _R, _R), :], xbufs[k], in_sems[k])

    def start_out(u):
        c, b = units[u]
        k = u % 2
        return pltpu.async_copy(
            xbufs[k], out_hbm.at[b, pl.ds(t0 + c * _R, _R), :], out_sems[k])

    def compute(u):
        k = u % 2
        buf = xbufs[k]

        @plsc.parallel_loop(0, _R * nslice, 1, unroll=8)
        def slice_body(i):
            r = i // nslice
            col = (i % nslice) * 16
            v = posbuf[r, pl.ds(col, 16)]
            plsc.addupdate(buf.at[r, pl.ds(col, 16)], v)

    in_dma = [None] * n_units
    out_dma = [None] * n_units
    in_dma[0] = start_in(0)
    for u in range(n_units):
        c, b = units[u]
        if b == 0:
            pltpu.sync_copy(pos_hbm.at[pl.ds(t0 + c * _R, _R), :], posbuf)
        if u + 1 < n_units:
            if u - 1 >= 0:
                out_dma[u - 1].wait()
            in_dma[u + 1] = start_in(u + 1)
        in_dma[u].wait()
        compute(u)
        out_dma[u] = start_out(u)
    out_dma[n_units - 1].wait()
    if n_units >= 2:
        out_dma[n_units - 2].wait()


def kernel(x, pos_embedding):
    B, T, D = x.shape
    mesh = plsc.VectorSubcoreMesh(core_axis_name="c", subcore_axis_name="s")
    f = functools.partial(
        pl.kernel,
        mesh=mesh,
        out_type=jax.ShapeDtypeStruct((B, T, D), x.dtype),
        scratch_types=[
            pltpu.VMEM((_R, D), jnp.float32),
            pltpu.VMEM((_R, D), jnp.float32),
            pltpu.VMEM((_R, D), jnp.float32),
            pltpu.SemaphoreType.DMA,
            pltpu.SemaphoreType.DMA,
            pltpu.SemaphoreType.DMA,
            pltpu.SemaphoreType.DMA,
        ],
    )(functools.partial(_sc_add_kernel, T))
    return f(x, pos_embedding)
